# Initial kernel scaffold; baseline (speedup 1.0000x reference)
import jax
import jax.numpy as jnp
from jax.experimental import pallas as pl


def kernel(x, edge_index, edge_attr, info_batch, W_g1, b_g1, bias_g1, W_g2, b_g2, bias_g2, nc_W1, nc_b1, nc_W2, nc_b2, ec_W1, ec_b1, ec_W2, ec_b2):
    raise NotImplementedError("write your pallas kernel here")


# speedup vs baseline: 4.8718x; 4.8718x over previous
import jax
import jax.numpy as jnp
from jax import lax
from jax.experimental import pallas as pl
from jax.experimental.pallas import tpu as pltpu
from jax.experimental.pallas import tpu_sc as plsc

NN = 10000
EE = 320000
EH = EE // 2
HH = 128
NB = 64

NC, NS = 2, 16
NW = NC * NS

NPAD16 = 10240
NPW = NPAD16 // NS
DEGW = 128

EEP = 327680
EGRPS = EEP // 128
EGPW = EGRPS // NW

EHP = 163840
EDGE_ROWS = EHP // 128
EDGE_RPW = EDGE_ROWS // NW
EPW = EHP // NW

_F32 = jnp.float32
_I32 = jnp.int32


def _sc_mesh():
  return plsc.VectorSubcoreMesh(
      core_axis_name="c", subcore_axis_name="s", num_cores=NC, num_subcores=NS)


def _deg_body(colg_hbm, out_hbm, colbuf, buf, acc):
  c = lax.axis_index("c")
  s = lax.axis_index("s")
  w = c * NS + s

  def f_fill(val):
    def f_i(i, _):
      def f_j(j, _):
        buf[i, pl.ds(j * 16, 16)] = jnp.full((16,), val, _F32)
        return 0
      lax.fori_loop(0, DEGW // 16, f_j, 0)
      return 0
    lax.fori_loop(0, 128, f_i, 0)

  f_fill(0.0)

  def f_zc(k, _):
    pltpu.sync_copy(buf, acc.at[pl.ds(s * NPW + k * 128, 128)])
    return 0
  lax.fori_loop(0, NPW // 128, f_zc, 0)

  f_fill(1.0)
  pltpu.sync_copy(colg_hbm.at[pl.ds(w * EGPW, EGPW)], colbuf)
  plsc.subcore_barrier()

  def f_grp(g, _):
    pltpu.sync_copy(buf, acc.at[colbuf.at[g]], add=True)
    return 0
  lax.fori_loop(0, EGPW, f_grp, 0)
  plsc.subcore_barrier()

  def f_out(k, _):
    base = s * NPW + k * 128
    pltpu.sync_copy(acc.at[pl.ds(base, 128)], buf)
    pltpu.sync_copy(buf, out_hbm.at[c, pl.ds(base, 128)])
    return 0
  lax.fori_loop(0, NPW // 128, f_out, 0)


def _deg_call(colg):
  k = pl.kernel(
      _deg_body,
      out_type=jax.ShapeDtypeStruct((NC, NPAD16, DEGW), _F32),
      mesh=_sc_mesh(),
      scratch_types=[
          pltpu.VMEM((EGPW, 128), _I32),
          pltpu.VMEM((128, DEGW), _F32),
          pltpu.VMEM_SHARED((NPAD16, DEGW), _F32),
      ],
  )
  return k(colg)


def _prop_body(y_hbm, rowg_hbm, colg_hbm, out_hbm, rowbuf, colbuf, gbuf, acc):
  c = lax.axis_index("c")
  s = lax.axis_index("s")
  w = c * NS + s

  def f_zero(i, _):
    def f_zj(j, _):
      gbuf[i, pl.ds(j * 16, 16)] = jnp.zeros((16,), _F32)
      return 0
    lax.fori_loop(0, HH // 16, f_zj, 0)
    return 0
  lax.fori_loop(0, 128, f_zero, 0)

  def f_zc(k, _):
    pltpu.sync_copy(gbuf, acc.at[pl.ds(s * NPW + k * 128, 128)])
    return 0
  lax.fori_loop(0, NPW // 128, f_zc, 0)

  pltpu.sync_copy(rowg_hbm.at[pl.ds(w * EGPW, EGPW)], rowbuf)
  pltpu.sync_copy(colg_hbm.at[pl.ds(w * EGPW, EGPW)], colbuf)
  plsc.subcore_barrier()

  def f_grp(g, _):
    pltpu.sync_copy(y_hbm.at[rowbuf.at[g]], gbuf)
    pltpu.sync_copy(gbuf, acc.at[colbuf.at[g]], add=True)
    return 0
  lax.fori_loop(0, EGPW, f_grp, 0)
  plsc.subcore_barrier()

  def f_out(k, _):
    base = s * NPW + k * 128
    pltpu.sync_copy(acc.at[pl.ds(base, 128)], gbuf)
    pltpu.sync_copy(gbuf, out_hbm.at[c, pl.ds(base, 128)])
    return 0
  lax.fori_loop(0, NPW // 128, f_out, 0)


def _prop_call(y, rowg, colg):
  k = pl.kernel(
      _prop_body,
      out_type=jax.ShapeDtypeStruct((NC, NPAD16, HH), _F32),
      mesh=_sc_mesh(),
      scratch_types=[
          pltpu.VMEM((EGPW, 128), _I32),
          pltpu.VMEM((EGPW, 128), _I32),
          pltpu.VMEM((128, HH), _F32),
          pltpu.VMEM_SHARED((NPAD16, HH), _F32),
      ],
  )
  return k(y, rowg, colg)


def _edge_body(a1_hbm, a2_hbm, b1_hbm, b2_hbm, srcg_hbm, dstg_hbm, ib_hbm,
               w2_hbm, eb2_hbm, el_hbm, ebatch_hbm, srcbuf, dstbuf, ab1, ab2,
               bb1, bb2, ibbuf, w2buf, eb2buf, elbuf, ebbuf):
  c = lax.axis_index("c")
  s = lax.axis_index("s")
  w = c * NS + s

  pltpu.sync_copy(srcg_hbm.at[pl.ds(w * EDGE_RPW, EDGE_RPW)], srcbuf)
  pltpu.sync_copy(dstg_hbm.at[pl.ds(w * EDGE_RPW, EDGE_RPW)], dstbuf)
  pltpu.sync_copy(ib_hbm, ibbuf)
  pltpu.sync_copy(w2_hbm, w2buf)
  pltpu.sync_copy(eb2_hbm, eb2buf)

  w2regs = [w2buf[pl.ds(j * 16, 16)] for j in range(16)]
  eb2vec = eb2buf[pl.ds(0, 16)]

  def f_grp(g, _):
    pltpu.sync_copy(a1_hbm.at[srcbuf.at[g]], ab1)
    pltpu.sync_copy(a2_hbm.at[srcbuf.at[g]], ab2)
    pltpu.sync_copy(b1_hbm.at[dstbuf.at[g]], bb1)
    pltpu.sync_copy(b2_hbm.at[dstbuf.at[g]], bb2)

    def f_chunk(t, _):
      eids = lax.iota(_I32, 16) + t * 16
      acc = eb2vec
      for k in range(2 * HH):
        atab, btab = (ab1, bb1) if k < HH else (ab2, bb2)
        kf = jnp.full((16,), k % HH, _I32)
        av = plsc.load_gather(atab, [eids, kf])
        bv = plsc.load_gather(btab, [eids, kf])
        w2k = w2regs[k // 16][k % 16]
        acc = acc + jnp.maximum(av + bv, 0.0) * w2k
      elbuf[pl.ds(g * 128 + t * 16, 16)] = acc
      return 0
    lax.fori_loop(0, 8, f_chunk, 0)

    def f_eb(i, _):
      sv = srcbuf[g, pl.ds(i * 16, 16)]
      ebbuf[pl.ds(g * 128 + i * 16, 16)] = plsc.load_gather(ibbuf, [sv])
      return 0
    lax.fori_loop(0, 8, f_eb, 0)
    return 0
  lax.fori_loop(0, EDGE_RPW, f_grp, 0)

  pltpu.sync_copy(elbuf, el_hbm.at[pl.ds(w * EPW, EPW)])
  pltpu.sync_copy(ebbuf, ebatch_hbm.at[pl.ds(w * EPW, EPW)])


def _edge_call(a1, a2, b1, b2, srcg, dstg, ib, w2, eb2):
  k = pl.kernel(
      _edge_body,
      out_type=(jax.ShapeDtypeStruct((EHP,), _F32),
                jax.ShapeDtypeStruct((EHP,), _I32)),
      mesh=_sc_mesh(),
      scratch_types=[
          pltpu.VMEM((EDGE_RPW, 128), _I32),
          pltpu.VMEM((EDGE_RPW, 128), _I32),
          pltpu.VMEM((128, HH), _F32),
          pltpu.VMEM((128, HH), _F32),
          pltpu.VMEM((128, HH), _F32),
          pltpu.VMEM((128, HH), _F32),
          pltpu.VMEM((NN,), _I32),
          pltpu.VMEM((2 * HH,), _F32),
          pltpu.VMEM((16,), _F32),
          pltpu.VMEM((EPW,), _F32),
          pltpu.VMEM((EPW,), _I32),
      ],
      compiler_params=pltpu.CompilerParams(needs_layout_passes=False),
  )
  return k(a1, a2, b1, b2, srcg, dstg, ib, w2, eb2)


_RB = 1000
_NBLK = NN // _RB


def _tc_b_body(degp, x, wt, b, y_out, dis_out):
  deg = degp[0, :, 0:1] + degp[1, :, 0:1] + 1.0
  dis = lax.rsqrt(deg)
  xw = jnp.dot(x[...], wt[...], preferred_element_type=_F32) + b[...]
  y_out[...] = dis * xw
  dis_out[...] = dis


def _tc_b_call(degp, x, wt, b):
  return pl.pallas_call(
      _tc_b_body,
      grid=(_NBLK,),
      in_specs=[
          pl.BlockSpec((NC, _RB, DEGW), lambda i: (0, i, 0)),
          pl.BlockSpec((_RB, HH), lambda i: (i, 0)),
          pl.BlockSpec((HH, HH), lambda i: (0, 0)),
          pl.BlockSpec((1, HH), lambda i: (0, 0)),
      ],
      out_specs=[
          pl.BlockSpec((_RB, HH), lambda i: (i, 0)),
          pl.BlockSpec((_RB, 1), lambda i: (i, 0)),
      ],
      out_shape=[
          jax.ShapeDtypeStruct((NN, HH), _F32),
          jax.ShapeDtypeStruct((NN, 1), _F32),
      ],
  )(degp, x, wt, b)


def _tc_d_body(accp, y1, dis, bias1, w2t, b2, y2_out):
  d = dis[...]
  h = d * (accp[0] + accp[1] + y1[...]) + bias1[...]
  y2_out[...] = d * (jnp.dot(h, w2t[...], preferred_element_type=_F32) + b2[...])


def _tc_d_call(accp, y1, dis, bias1, w2t, b2):
  return pl.pallas_call(
      _tc_d_body,
      grid=(_NBLK,),
      in_specs=[
          pl.BlockSpec((NC, _RB, HH), lambda i: (0, i, 0)),
          pl.BlockSpec((_RB, HH), lambda i: (i, 0)),
          pl.BlockSpec((_RB, 1), lambda i: (i, 0)),
          pl.BlockSpec((1, HH), lambda i: (0, 0)),
          pl.BlockSpec((HH, HH), lambda i: (0, 0)),
          pl.BlockSpec((1, HH), lambda i: (0, 0)),
      ],
      out_specs=pl.BlockSpec((_RB, HH), lambda i: (i, 0)),
      out_shape=jax.ShapeDtypeStruct((NN, HH), _F32),
  )(accp, y1, dis, bias1, w2t, b2)


def _tc_f_body(accp, y2, dis, bias2, ncw1t, ncb1, ncw2c, ncb2, wa1, wa2, wb1,
               wb2, ecb1a, ecb1b, nl_out, a1_out, a2_out, b1_out, b2_out):
  d = dis[...]
  h = d * (accp[0] + accp[1] + y2[...]) + bias2[...]
  t = jnp.maximum(
      jnp.dot(h, ncw1t[...], preferred_element_type=_F32) + ncb1[...], 0.0)
  nl_out[...] = jnp.dot(t, ncw2c[...], preferred_element_type=_F32) + ncb2[...]
  a1_out[...] = jnp.dot(h, wa1[...], preferred_element_type=_F32) + ecb1a[...]
  a2_out[...] = jnp.dot(h, wa2[...], preferred_element_type=_F32) + ecb1b[...]
  b1_out[...] = jnp.dot(h, wb1[...], preferred_element_type=_F32)
  b2_out[...] = jnp.dot(h, wb2[...], preferred_element_type=_F32)


def _tc_f_call(accp, y2, dis, bias2, ncw1t, ncb1, ncw2c, ncb2, wa1, wa2, wb1,
               wb2, ecb1a, ecb1b):
  full = lambda shp: pl.BlockSpec(shp, lambda i: tuple(0 for _ in shp))
  rowblk = pl.BlockSpec((_RB, HH), lambda i: (i, 0))
  return pl.pallas_call(
      _tc_f_body,
      grid=(_NBLK,),
      in_specs=[
          pl.BlockSpec((NC, _RB, HH), lambda i: (0, i, 0)),
          rowblk,
          pl.BlockSpec((_RB, 1), lambda i: (i, 0)),
          full((1, HH)),
          full((HH, HH)),
          full((1, HH)),
          full((HH, 1)),
          full((1, 1)),
          full((HH, HH)),
          full((HH, HH)),
          full((HH, HH)),
          full((HH, HH)),
          full((1, HH)),
          full((1, HH)),
      ],
      out_specs=[
          pl.BlockSpec((_RB, 1), lambda i: (i, 0)),
          rowblk, rowblk, rowblk, rowblk,
      ],
      out_shape=[
          jax.ShapeDtypeStruct((NN, 1), _F32),
          jax.ShapeDtypeStruct((NN, HH), _F32),
          jax.ShapeDtypeStruct((NN, HH), _F32),
          jax.ShapeDtypeStruct((NN, HH), _F32),
          jax.ShapeDtypeStruct((NN, HH), _F32),
      ],
  )(accp, y2, dis, bias2, ncw1t, ncb1, ncw2c, ncb2, wa1, wa2, wb1, wb2,
    ecb1a, ecb1b)


_NROWS = NPAD16 // 128
_EROWS = EH // 128


def _sm_body(nl2, ib2, el2, eb2, on_out, oe_out):
  nl = nl2[...]
  el = el2[...]
  ib = ib2[...]
  eb = eb2[...]
  m = jnp.maximum(jnp.max(nl), jnp.max(el))
  evn = jnp.exp(nl - m)
  eve = jnp.exp(el - m)
  svals = []
  for b in range(NB):
    sb = (jnp.sum(jnp.where(ib == b, evn, 0.0)) +
          jnp.sum(jnp.where(eb == b, eve, 0.0)))
    svals.append(sb)
  rsn = jnp.zeros_like(nl)
  rse = jnp.zeros_like(el)
  for b in range(NB):
    r = 1.0 / svals[b]
    rsn = rsn + jnp.where(ib == b, r, 0.0)
    rse = rse + jnp.where(eb == b, r, 0.0)
  on_out[...] = evn * rsn
  oe_out[...] = eve * rse


def _sm_call(nl2, ib2, el2, eb2):
  return pl.pallas_call(
      _sm_body,
      out_shape=[
          jax.ShapeDtypeStruct((_NROWS, 128), _F32),
          jax.ShapeDtypeStruct((_EROWS, 128), _F32),
      ],
  )(nl2, ib2, el2, eb2)


def kernel(x, edge_index, edge_attr, info_batch,
           W_g1, b_g1, bias_g1, W_g2, b_g2, bias_g2,
           nc_W1, nc_b1, nc_W2, nc_b2, ec_W1, ec_b1, ec_W2, ec_b2):
  npad = EEP - EE
  rowp = (jnp.arange(npad, dtype=_I32) % 16)
  colp = NN + (jnp.arange(npad, dtype=_I32) % 128)
  rowg = jnp.concatenate([edge_index[0], rowp]).reshape(EGRPS, 128)
  colg = jnp.concatenate([edge_index[1], colp]).reshape(EGRPS, 128)

  degp = _deg_call(colg)

  y1, dis = _tc_b_call(degp, x, W_g1.T, b_g1.reshape(1, HH))
  acc1 = _prop_call(y1, rowg, colg)
  acc1 = _prop_call(y1, rowg, colg)
  y2 = _tc_d_call(acc1, y1, dis, bias_g1.reshape(1, HH), W_g2.T,
                  b_g2.reshape(1, HH))
  acc2 = _prop_call(y2, rowg, colg)

  w1t = ec_W1.T
  nl, a1, a2, b1, b2 = _tc_f_call(
      acc2, y2, dis, bias_g2.reshape(1, HH),
      nc_W1.T, nc_b1.reshape(1, HH), nc_W2.T, nc_b2.reshape(1, 1),
      w1t[:HH, :HH], w1t[:HH, HH:], w1t[HH:, :HH], w1t[HH:, HH:],
      ec_b1[:HH].reshape(1, HH), ec_b1[HH:].reshape(1, HH))

  src = edge_index[0, ::2]
  dst = edge_index[1, ::2]
  padi = (jnp.arange(EHP - EH, dtype=_I32) % 16)
  srcg = jnp.concatenate([src, padi]).reshape(EDGE_ROWS, 128)
  dstg = jnp.concatenate([dst, padi]).reshape(EDGE_ROWS, 128)
  w2 = ec_W2.reshape(2 * HH)
  eb2c = jnp.broadcast_to(ec_b2.reshape(1), (16,))

  elp, ebp = _edge_call(a1, a2, b1, b2, srcg, dstg, info_batch, w2, eb2c)

  nlpad = jnp.concatenate(
      [nl.reshape(NN),
       jnp.full((NPAD16 - NN,), -1e30, _F32)]).reshape(_NROWS, 128)
  ibpad = jnp.concatenate(
      [info_batch, jnp.full((NPAD16 - NN,), NB - 1, _I32)]).reshape(_NROWS, 128)
  el2 = elp[:EH].reshape(_EROWS, 128)
  eb2d = ebp[:EH].reshape(_EROWS, 128)

  on, oe = _sm_call(nlpad, ibpad, el2, eb2d)

  return (on.reshape(NPAD16)[:NN].reshape(NN, 1),
          oe.reshape(EH, 1))


# speedup vs baseline: 5.1718x; 1.0616x over previous
import jax
import jax.numpy as jnp
from jax import lax
from jax.experimental import pallas as pl
from jax.experimental.pallas import tpu as pltpu
from jax.experimental.pallas import tpu_sc as plsc

NN = 10000
EE = 320000
EH = EE // 2
HH = 128
NB = 64

NC, NS = 2, 16
NW = NC * NS

NPAD16 = 10240
NPW = NPAD16 // NS
DEGW = 128

EEP = 327680
EGRPS = EEP // 128
EGPW = EGRPS // NW

EHP = 163840
EDGE_ROWS = EHP // 128
EDGE_RPW = EDGE_ROWS // NW
EPW = EHP // NW

_F32 = jnp.float32
_I32 = jnp.int32


def _sc_mesh():
  return plsc.VectorSubcoreMesh(
      core_axis_name="c", subcore_axis_name="s", num_cores=NC, num_subcores=NS)


def _deg_body(colg_hbm, out_hbm, colbuf, buf, acc):
  c = lax.axis_index("c")
  s = lax.axis_index("s")
  w = c * NS + s

  def f_fill(val):
    def f_i(i, _):
      def f_j(j, _):
        buf[i, pl.ds(j * 16, 16)] = jnp.full((16,), val, _F32)
        return 0
      lax.fori_loop(0, DEGW // 16, f_j, 0)
      return 0
    lax.fori_loop(0, 128, f_i, 0)

  f_fill(0.0)

  def f_zc(k, _):
    pltpu.sync_copy(buf, acc.at[pl.ds(s * NPW + k * 128, 128)])
    return 0
  lax.fori_loop(0, NPW // 128, f_zc, 0)

  f_fill(1.0)
  pltpu.sync_copy(colg_hbm.at[pl.ds(w * EGPW, EGPW)], colbuf)
  plsc.subcore_barrier()

  def f_grp(g, _):
    pltpu.sync_copy(buf, acc.at[colbuf.at[g]], add=True)
    return 0
  lax.fori_loop(0, EGPW, f_grp, 0)
  plsc.subcore_barrier()

  def f_out(k, _):
    base = s * NPW + k * 128
    pltpu.sync_copy(acc.at[pl.ds(base, 128)], buf)
    pltpu.sync_copy(buf, out_hbm.at[c, pl.ds(base, 128)])
    return 0
  lax.fori_loop(0, NPW // 128, f_out, 0)


def _deg_call(colg):
  k = pl.kernel(
      _deg_body,
      out_type=jax.ShapeDtypeStruct((NC, NPAD16, DEGW), _F32),
      mesh=_sc_mesh(),
      scratch_types=[
          pltpu.VMEM((EGPW, 128), _I32),
          pltpu.VMEM((128, DEGW), _F32),
          pltpu.VMEM_SHARED((NPAD16, DEGW), _F32),
      ],
  )
  return k(colg)


_HG = EGPW // 2


def _prop_body(y_hbm, rowg_hbm, colg_hbm, out_hbm, rowbuf, colbuf, gbuf0,
               gbuf1, sem0, sem1, acc):
  c = lax.axis_index("c")
  s = lax.axis_index("s")
  w = c * NS + s

  def f_zero(i, _):
    def f_zj(j, _):
      gbuf0[i, pl.ds(j * 16, 16)] = jnp.zeros((16,), _F32)
      return 0
    lax.fori_loop(0, HH // 16, f_zj, 0)
    return 0
  lax.fori_loop(0, 128, f_zero, 0)

  def f_zc(k, _):
    pltpu.sync_copy(gbuf0, acc.at[pl.ds(s * NPW + k * 128, 128)])
    return 0
  lax.fori_loop(0, NPW // 128, f_zc, 0)
  plsc.subcore_barrier()

  def w0(g):
    return pltpu.make_async_copy(y_hbm.at[rowbuf.at[g]], gbuf0, sem0).wait()

  def w1(g):
    return pltpu.make_async_copy(y_hbm.at[rowbuf.at[g]], gbuf1, sem1).wait()

  for h in range(2):
    pltpu.sync_copy(rowg_hbm.at[pl.ds(w * EGPW + h * _HG, _HG)], rowbuf)
    pltpu.sync_copy(colg_hbm.at[pl.ds(w * EGPW + h * _HG, _HG)], colbuf)
    pltpu.async_copy(y_hbm.at[rowbuf.at[0]], gbuf0, sem0)

    def f_grp(g2, _):
      g = g2 * 2
      pltpu.async_copy(y_hbm.at[rowbuf.at[g + 1]], gbuf1, sem1)
      w0(g)
      pltpu.sync_copy(gbuf0, acc.at[colbuf.at[g]], add=True)
      pltpu.async_copy(y_hbm.at[rowbuf.at[g + 2]], gbuf0, sem0)
      w1(g + 1)
      pltpu.sync_copy(gbuf1, acc.at[colbuf.at[g + 1]], add=True)
      return 0
    lax.fori_loop(0, _HG // 2 - 1, f_grp, 0)

    ge = _HG - 2
    pltpu.async_copy(y_hbm.at[rowbuf.at[ge + 1]], gbuf1, sem1)
    w0(ge)
    pltpu.sync_copy(gbuf0, acc.at[colbuf.at[ge]], add=True)
    w1(ge + 1)
    pltpu.sync_copy(gbuf1, acc.at[colbuf.at[ge + 1]], add=True)

  plsc.subcore_barrier()

  def f_out(k, _):
    base = s * NPW + k * 128
    pltpu.sync_copy(acc.at[pl.ds(base, 128)], gbuf0)
    pltpu.sync_copy(gbuf0, out_hbm.at[c, pl.ds(base, 128)])
    return 0
  lax.fori_loop(0, NPW // 128, f_out, 0)


def _prop_call(y, rowg, colg):
  k = pl.kernel(
      _prop_body,
      out_type=jax.ShapeDtypeStruct((NC, NPAD16, HH), _F32),
      mesh=_sc_mesh(),
      scratch_types=[
          pltpu.VMEM((_HG, 128), _I32),
          pltpu.VMEM((_HG, 128), _I32),
          pltpu.VMEM((128, HH), _F32),
          pltpu.VMEM((128, HH), _F32),
          pltpu.SemaphoreType.DMA,
          pltpu.SemaphoreType.DMA,
          pltpu.VMEM_SHARED((NPAD16, HH), _F32),
      ],
  )
  return k(y, rowg, colg)


def _edge_body(a1_hbm, a2_hbm, b1_hbm, b2_hbm, srcg_hbm, dstg_hbm, ib_hbm,
               w2_hbm, eb2_hbm, el_hbm, ebatch_hbm, srcbuf, dstbuf, ab1, ab2,
               bb1, bb2, ibbuf, w2buf, eb2buf, elbuf, ebbuf):
  c = lax.axis_index("c")
  s = lax.axis_index("s")
  w = c * NS + s

  pltpu.sync_copy(srcg_hbm.at[pl.ds(w * EDGE_RPW, EDGE_RPW)], srcbuf)
  pltpu.sync_copy(dstg_hbm.at[pl.ds(w * EDGE_RPW, EDGE_RPW)], dstbuf)
  pltpu.sync_copy(ib_hbm, ibbuf)
  pltpu.sync_copy(w2_hbm, w2buf)
  pltpu.sync_copy(eb2_hbm, eb2buf)

  w2regs = [w2buf[pl.ds(j * 16, 16)] for j in range(16)]
  eb2vec = eb2buf[pl.ds(0, 16)]

  def f_grp(g, _):
    pltpu.sync_copy(a1_hbm.at[srcbuf.at[g]], ab1)
    pltpu.sync_copy(a2_hbm.at[srcbuf.at[g]], ab2)
    pltpu.sync_copy(b1_hbm.at[dstbuf.at[g]], bb1)
    pltpu.sync_copy(b2_hbm.at[dstbuf.at[g]], bb2)

    def f_chunk(t, _):
      eids = lax.iota(_I32, 16) + t * 16
      acc = eb2vec
      for k in range(2 * HH):
        atab, btab = (ab1, bb1) if k < HH else (ab2, bb2)
        kf = jnp.full((16,), k % HH, _I32)
        av = plsc.load_gather(atab, [eids, kf])
        bv = plsc.load_gather(btab, [eids, kf])
        w2k = w2regs[k // 16][k % 16]
        acc = acc + jnp.maximum(av + bv, 0.0) * w2k
      elbuf[pl.ds(g * 128 + t * 16, 16)] = acc
      return 0
    lax.fori_loop(0, 8, f_chunk, 0)

    def f_eb(i, _):
      sv = srcbuf[g, pl.ds(i * 16, 16)]
      ebbuf[pl.ds(g * 128 + i * 16, 16)] = plsc.load_gather(ibbuf, [sv])
      return 0
    lax.fori_loop(0, 8, f_eb, 0)
    return 0
  lax.fori_loop(0, EDGE_RPW, f_grp, 0)

  pltpu.sync_copy(elbuf, el_hbm.at[pl.ds(w * EPW, EPW)])
  pltpu.sync_copy(ebbuf, ebatch_hbm.at[pl.ds(w * EPW, EPW)])


def _edge_call(a1, a2, b1, b2, srcg, dstg, ib, w2, eb2):
  k = pl.kernel(
      _edge_body,
      out_type=(jax.ShapeDtypeStruct((EHP,), _F32),
                jax.ShapeDtypeStruct((EHP,), _I32)),
      mesh=_sc_mesh(),
      scratch_types=[
          pltpu.VMEM((EDGE_RPW, 128), _I32),
          pltpu.VMEM((EDGE_RPW, 128), _I32),
          pltpu.VMEM((128, HH), _F32),
          pltpu.VMEM((128, HH), _F32),
          pltpu.VMEM((128, HH), _F32),
          pltpu.VMEM((128, HH), _F32),
          pltpu.VMEM((NN,), _I32),
          pltpu.VMEM((2 * HH,), _F32),
          pltpu.VMEM((16,), _F32),
          pltpu.VMEM((EPW,), _F32),
          pltpu.VMEM((EPW,), _I32),
      ],
      compiler_params=pltpu.CompilerParams(needs_layout_passes=False),
  )
  return k(a1, a2, b1, b2, srcg, dstg, ib, w2, eb2)


_RB = 1000
_NBLK = NN // _RB


def _tc_b_body(degp, x, wt, b, y_out, dis_out):
  deg = degp[0, :, 0:1] + degp[1, :, 0:1] + 1.0
  dis = lax.rsqrt(deg)
  xw = jnp.dot(x[...], wt[...], preferred_element_type=_F32) + b[...]
  y_out[...] = dis * xw
  dis_out[...] = dis


def _tc_b_call(degp, x, wt, b):
  return pl.pallas_call(
      _tc_b_body,
      grid=(_NBLK,),
      in_specs=[
          pl.BlockSpec((NC, _RB, DEGW), lambda i: (0, i, 0)),
          pl.BlockSpec((_RB, HH), lambda i: (i, 0)),
          pl.BlockSpec((HH, HH), lambda i: (0, 0)),
          pl.BlockSpec((1, HH), lambda i: (0, 0)),
      ],
      out_specs=[
          pl.BlockSpec((_RB, HH), lambda i: (i, 0)),
          pl.BlockSpec((_RB, 1), lambda i: (i, 0)),
      ],
      out_shape=[
          jax.ShapeDtypeStruct((NN, HH), _F32),
          jax.ShapeDtypeStruct((NN, 1), _F32),
      ],
  )(degp, x, wt, b)


def _tc_d_body(accp, y1, dis, bias1, w2t, b2, y2_out):
  d = dis[...]
  h = d * (accp[0] + accp[1] + y1[...]) + bias1[...]
  y2_out[...] = d * (jnp.dot(h, w2t[...], preferred_element_type=_F32) + b2[...])


def _tc_d_call(accp, y1, dis, bias1, w2t, b2):
  return pl.pallas_call(
      _tc_d_body,
      grid=(_NBLK,),
      in_specs=[
          pl.BlockSpec((NC, _RB, HH), lambda i: (0, i, 0)),
          pl.BlockSpec((_RB, HH), lambda i: (i, 0)),
          pl.BlockSpec((_RB, 1), lambda i: (i, 0)),
          pl.BlockSpec((1, HH), lambda i: (0, 0)),
          pl.BlockSpec((HH, HH), lambda i: (0, 0)),
          pl.BlockSpec((1, HH), lambda i: (0, 0)),
      ],
      out_specs=pl.BlockSpec((_RB, HH), lambda i: (i, 0)),
      out_shape=jax.ShapeDtypeStruct((NN, HH), _F32),
  )(accp, y1, dis, bias1, w2t, b2)


def _tc_f_body(accp, y2, dis, bias2, ncw1t, ncb1, ncw2c, ncb2, wa1, wa2, wb1,
               wb2, ecb1a, ecb1b, nl_out, a1_out, a2_out, b1_out, b2_out):
  d = dis[...]
  h = d * (accp[0] + accp[1] + y2[...]) + bias2[...]
  t = jnp.maximum(
      jnp.dot(h, ncw1t[...], preferred_element_type=_F32) + ncb1[...], 0.0)
  nl_out[...] = jnp.dot(t, ncw2c[...], preferred_element_type=_F32) + ncb2[...]
  a1_out[...] = jnp.dot(h, wa1[...], preferred_element_type=_F32) + ecb1a[...]
  a2_out[...] = jnp.dot(h, wa2[...], preferred_element_type=_F32) + ecb1b[...]
  b1_out[...] = jnp.dot(h, wb1[...], preferred_element_type=_F32)
  b2_out[...] = jnp.dot(h, wb2[...], preferred_element_type=_F32)


def _tc_f_call(accp, y2, dis, bias2, ncw1t, ncb1, ncw2c, ncb2, wa1, wa2, wb1,
               wb2, ecb1a, ecb1b):
  full = lambda shp: pl.BlockSpec(shp, lambda i: tuple(0 for _ in shp))
  rowblk = pl.BlockSpec((_RB, HH), lambda i: (i, 0))
  return pl.pallas_call(
      _tc_f_body,
      grid=(_NBLK,),
      in_specs=[
          pl.BlockSpec((NC, _RB, HH), lambda i: (0, i, 0)),
          rowblk,
          pl.BlockSpec((_RB, 1), lambda i: (i, 0)),
          full((1, HH)),
          full((HH, HH)),
          full((1, HH)),
          full((HH, 1)),
          full((1, 1)),
          full((HH, HH)),
          full((HH, HH)),
          full((HH, HH)),
          full((HH, HH)),
          full((1, HH)),
          full((1, HH)),
      ],
      out_specs=[
          pl.BlockSpec((_RB, 1), lambda i: (i, 0)),
          rowblk, rowblk, rowblk, rowblk,
      ],
      out_shape=[
          jax.ShapeDtypeStruct((NN, 1), _F32),
          jax.ShapeDtypeStruct((NN, HH), _F32),
          jax.ShapeDtypeStruct((NN, HH), _F32),
          jax.ShapeDtypeStruct((NN, HH), _F32),
          jax.ShapeDtypeStruct((NN, HH), _F32),
      ],
  )(accp, y2, dis, bias2, ncw1t, ncb1, ncw2c, ncb2, wa1, wa2, wb1, wb2,
    ecb1a, ecb1b)


_NROWS = NPAD16 // 128
_EROWS = EH // 128


def _sm_body(nl2, ib2, el2, eb2, on_out, oe_out):
  nl = nl2[...]
  el = el2[...]
  ib = ib2[...]
  eb = eb2[...]
  m = jnp.maximum(jnp.max(nl), jnp.max(el))
  evn = jnp.exp(nl - m)
  eve = jnp.exp(el - m)
  svals = []
  for b in range(NB):
    sb = (jnp.sum(jnp.where(ib == b, evn, 0.0)) +
          jnp.sum(jnp.where(eb == b, eve, 0.0)))
    svals.append(sb)
  rsn = jnp.zeros_like(nl)
  rse = jnp.zeros_like(el)
  for b in range(NB):
    r = 1.0 / svals[b]
    rsn = rsn + jnp.where(ib == b, r, 0.0)
    rse = rse + jnp.where(eb == b, r, 0.0)
  on_out[...] = evn * rsn
  oe_out[...] = eve * rse


def _sm_call(nl2, ib2, el2, eb2):
  return pl.pallas_call(
      _sm_body,
      out_shape=[
          jax.ShapeDtypeStruct((_NROWS, 128), _F32),
          jax.ShapeDtypeStruct((_EROWS, 128), _F32),
      ],
  )(nl2, ib2, el2, eb2)


def kernel(x, edge_index, edge_attr, info_batch,
           W_g1, b_g1, bias_g1, W_g2, b_g2, bias_g2,
           nc_W1, nc_b1, nc_W2, nc_b2, ec_W1, ec_b1, ec_W2, ec_b2):
  npad = EEP - EE
  rowp = (jnp.arange(npad, dtype=_I32) % 16)
  colp = NN + (jnp.arange(npad, dtype=_I32) % 128)
  rowg = jnp.concatenate([edge_index[0], rowp]).reshape(EGRPS, 128)
  colg = jnp.concatenate([edge_index[1], colp]).reshape(EGRPS, 128)

  degp = _deg_call(colg)

  y1, dis = _tc_b_call(degp, x, W_g1.T, b_g1.reshape(1, HH))
  acc1 = _prop_call(y1, rowg, colg)
  acc1 = _prop_call(y1, rowg, colg)
  y2 = _tc_d_call(acc1, y1, dis, bias_g1.reshape(1, HH), W_g2.T,
                  b_g2.reshape(1, HH))
  acc2 = _prop_call(y2, rowg, colg)

  w1t = ec_W1.T
  nl, a1, a2, b1, b2 = _tc_f_call(
      acc2, y2, dis, bias_g2.reshape(1, HH),
      nc_W1.T, nc_b1.reshape(1, HH), nc_W2.T, nc_b2.reshape(1, 1),
      w1t[:HH, :HH], w1t[:HH, HH:], w1t[HH:, :HH], w1t[HH:, HH:],
      ec_b1[:HH].reshape(1, HH), ec_b1[HH:].reshape(1, HH))

  src = edge_index[0, ::2]
  dst = edge_index[1, ::2]
  padi = (jnp.arange(EHP - EH, dtype=_I32) % 16)
  srcg = jnp.concatenate([src, padi]).reshape(EDGE_ROWS, 128)
  dstg = jnp.concatenate([dst, padi]).reshape(EDGE_ROWS, 128)
  w2 = ec_W2.reshape(2 * HH)
  eb2c = jnp.broadcast_to(ec_b2.reshape(1), (16,))

  elp, ebp = _edge_call(a1, a2, b1, b2, srcg, dstg, info_batch, w2, eb2c)

  nlpad = jnp.concatenate(
      [nl.reshape(NN),
       jnp.full((NPAD16 - NN,), -1e30, _F32)]).reshape(_NROWS, 128)
  ibpad = jnp.concatenate(
      [info_batch, jnp.full((NPAD16 - NN,), NB - 1, _I32)]).reshape(_NROWS, 128)
  el2 = elp[:EH].reshape(_EROWS, 128)
  eb2d = ebp[:EH].reshape(_EROWS, 128)

  on, oe = _sm_call(nlpad, ibpad, el2, eb2d)

  return (on.reshape(NPAD16)[:NN].reshape(NN, 1),
          oe.reshape(EH, 1))


# speedup vs baseline: 5.2628x; 1.0176x over previous
import jax
import jax.numpy as jnp
from jax import lax
from jax.experimental import pallas as pl
from jax.experimental.pallas import tpu as pltpu
from jax.experimental.pallas import tpu_sc as plsc

NN = 10000
EE = 320000
EH = EE // 2
HH = 128
NB = 64

NC, NS = 2, 16
NW = NC * NS

NPAD16 = 10240
NPW = NPAD16 // NS
DEGW = 32

EEP = 327680
EGRPS = EEP // 128
EGPW = EGRPS // NW

EHP = 163840
EDGE_ROWS = EHP // 128
EDGE_RPW = EDGE_ROWS // NW
EPW = EHP // NW

_F32 = jnp.float32
_I32 = jnp.int32


def _sc_mesh():
  return plsc.VectorSubcoreMesh(
      core_axis_name="c", subcore_axis_name="s", num_cores=NC, num_subcores=NS)


def _deg_body(colg_hbm, out_hbm, colbuf, buf, acc):
  c = lax.axis_index("c")
  s = lax.axis_index("s")
  w = c * NS + s

  def f_fill(val):
    def f_i(i, _):
      def f_j(j, _):
        buf[i, pl.ds(j * 16, 16)] = jnp.full((16,), val, _F32)
        return 0
      lax.fori_loop(0, DEGW // 16, f_j, 0)
      return 0
    lax.fori_loop(0, 128, f_i, 0)

  f_fill(0.0)

  def f_zc(k, _):
    pltpu.sync_copy(buf, acc.at[pl.ds(s * NPW + k * 128, 128)])
    return 0
  lax.fori_loop(0, NPW // 128, f_zc, 0)

  f_fill(1.0)
  pltpu.sync_copy(colg_hbm.at[pl.ds(w * EGPW, EGPW)], colbuf)
  plsc.subcore_barrier()

  def f_grp(g, _):
    pltpu.sync_copy(buf, acc.at[colbuf.at[g]], add=True)
    return 0
  lax.fori_loop(0, EGPW, f_grp, 0)
  plsc.subcore_barrier()

  def f_out(k, _):
    base = s * NPW + k * 128
    pltpu.sync_copy(acc.at[pl.ds(base, 128)], buf)
    pltpu.sync_copy(buf, out_hbm.at[c, pl.ds(base, 128)])
    return 0
  lax.fori_loop(0, NPW // 128, f_out, 0)


def _deg_call(colg):
  k = pl.kernel(
      _deg_body,
      out_type=jax.ShapeDtypeStruct((NC, NPAD16, DEGW), _F32),
      mesh=_sc_mesh(),
      scratch_types=[
          pltpu.VMEM((EGPW, 128), _I32),
          pltpu.VMEM((128, DEGW), _F32),
          pltpu.VMEM_SHARED((NPAD16, DEGW), _F32),
      ],
  )
  return k(colg)


_HG = EGPW // 2


def _prop_body(y_hbm, rowg_hbm, colg_hbm, out_hbm, rowbuf, colbuf, gbuf0,
               gbuf1, sem0, sem1, acc):
  c = lax.axis_index("c")
  s = lax.axis_index("s")
  w = c * NS + s

  def f_zero(i, _):
    def f_zj(j, _):
      gbuf0[i, pl.ds(j * 16, 16)] = jnp.zeros((16,), _F32)
      return 0
    lax.fori_loop(0, HH // 16, f_zj, 0)
    return 0
  lax.fori_loop(0, 128, f_zero, 0)

  def f_zc(k, _):
    pltpu.sync_copy(gbuf0, acc.at[pl.ds(s * NPW + k * 128, 128)])
    return 0
  lax.fori_loop(0, NPW // 128, f_zc, 0)
  plsc.subcore_barrier()

  def w0(g):
    return pltpu.make_async_copy(y_hbm.at[rowbuf.at[g]], gbuf0, sem0).wait()

  def w1(g):
    return pltpu.make_async_copy(y_hbm.at[rowbuf.at[g]], gbuf1, sem1).wait()

  for h in range(2):
    pltpu.sync_copy(rowg_hbm.at[pl.ds(w * EGPW + h * _HG, _HG)], rowbuf)
    pltpu.sync_copy(colg_hbm.at[pl.ds(w * EGPW + h * _HG, _HG)], colbuf)
    pltpu.async_copy(y_hbm.at[rowbuf.at[0]], gbuf0, sem0)

    def f_grp(g2, _):
      g = g2 * 2
      pltpu.async_copy(y_hbm.at[rowbuf.at[g + 1]], gbuf1, sem1)
      w0(g)
      pltpu.sync_copy(gbuf0, acc.at[colbuf.at[g]], add=True)
      pltpu.async_copy(y_hbm.at[rowbuf.at[g + 2]], gbuf0, sem0)
      w1(g + 1)
      pltpu.sync_copy(gbuf1, acc.at[colbuf.at[g + 1]], add=True)
      return 0
    lax.fori_loop(0, _HG // 2 - 1, f_grp, 0)

    ge = _HG - 2
    pltpu.async_copy(y_hbm.at[rowbuf.at[ge + 1]], gbuf1, sem1)
    w0(ge)
    pltpu.sync_copy(gbuf0, acc.at[colbuf.at[ge]], add=True)
    w1(ge + 1)
    pltpu.sync_copy(gbuf1, acc.at[colbuf.at[ge + 1]], add=True)

  plsc.subcore_barrier()

  def f_out(k, _):
    base = s * NPW + k * 128
    pltpu.sync_copy(acc.at[pl.ds(base, 128)], gbuf0)
    pltpu.sync_copy(gbuf0, out_hbm.at[c, pl.ds(base, 128)])
    return 0
  lax.fori_loop(0, NPW // 128, f_out, 0)


def _prop_call(y, rowg, colg):
  k = pl.kernel(
      _prop_body,
      out_type=jax.ShapeDtypeStruct((NC, NPAD16, HH), _F32),
      mesh=_sc_mesh(),
      scratch_types=[
          pltpu.VMEM((_HG, 128), _I32),
          pltpu.VMEM((_HG, 128), _I32),
          pltpu.VMEM((128, HH), _F32),
          pltpu.VMEM((128, HH), _F32),
          pltpu.SemaphoreType.DMA,
          pltpu.SemaphoreType.DMA,
          pltpu.VMEM_SHARED((NPAD16, HH), _F32),
      ],
  )
  return k(y, rowg, colg)


def _edge_body(a1_hbm, a2_hbm, b1_hbm, b2_hbm, srcg_hbm, dstg_hbm, ib_hbm,
               w2_hbm, eb2_hbm, el_hbm, ebatch_hbm, srcbuf, dstbuf, ab1, ab2,
               bb1, bb2, ibbuf, w2buf, eb2buf, elbuf, ebbuf):
  c = lax.axis_index("c")
  s = lax.axis_index("s")
  w = c * NS + s

  pltpu.sync_copy(srcg_hbm.at[pl.ds(w * EDGE_RPW, EDGE_RPW)], srcbuf)
  pltpu.sync_copy(dstg_hbm.at[pl.ds(w * EDGE_RPW, EDGE_RPW)], dstbuf)
  pltpu.sync_copy(ib_hbm, ibbuf)
  pltpu.sync_copy(w2_hbm, w2buf)
  pltpu.sync_copy(eb2_hbm, eb2buf)

  w2regs = [w2buf[pl.ds(j * 16, 16)] for j in range(16)]
  eb2vec = eb2buf[pl.ds(0, 16)]

  def f_grp(g, _):
    pltpu.sync_copy(a1_hbm.at[srcbuf.at[g]], ab1)
    pltpu.sync_copy(a2_hbm.at[srcbuf.at[g]], ab2)
    pltpu.sync_copy(b1_hbm.at[dstbuf.at[g]], bb1)
    pltpu.sync_copy(b2_hbm.at[dstbuf.at[g]], bb2)

    def f_chunk(t, _):
      eids = lax.iota(_I32, 16) + t * 16
      acc = eb2vec
      for k in range(2 * HH):
        atab, btab = (ab1, bb1) if k < HH else (ab2, bb2)
        kf = jnp.full((16,), k % HH, _I32)
        av = plsc.load_gather(atab, [eids, kf])
        bv = plsc.load_gather(btab, [eids, kf])
        w2k = w2regs[k // 16][k % 16]
        acc = acc + jnp.maximum(av + bv, 0.0) * w2k
      elbuf[pl.ds(g * 128 + t * 16, 16)] = acc
      return 0
    lax.fori_loop(0, 8, f_chunk, 0)

    def f_eb(i, _):
      sv = srcbuf[g, pl.ds(i * 16, 16)]
      ebbuf[pl.ds(g * 128 + i * 16, 16)] = plsc.load_gather(ibbuf, [sv])
      return 0
    lax.fori_loop(0, 8, f_eb, 0)
    return 0
  lax.fori_loop(0, EDGE_RPW, f_grp, 0)

  pltpu.sync_copy(elbuf, el_hbm.at[pl.ds(w * EPW, EPW)])
  pltpu.sync_copy(ebbuf, ebatch_hbm.at[pl.ds(w * EPW, EPW)])


def _edge_call(a1, a2, b1, b2, srcg, dstg, ib, w2, eb2):
  k = pl.kernel(
      _edge_body,
      out_type=(jax.ShapeDtypeStruct((EHP,), _F32),
                jax.ShapeDtypeStruct((EHP,), _I32)),
      mesh=_sc_mesh(),
      scratch_types=[
          pltpu.VMEM((EDGE_RPW, 128), _I32),
          pltpu.VMEM((EDGE_RPW, 128), _I32),
          pltpu.VMEM((128, HH), _F32),
          pltpu.VMEM((128, HH), _F32),
          pltpu.VMEM((128, HH), _F32),
          pltpu.VMEM((128, HH), _F32),
          pltpu.VMEM((NN,), _I32),
          pltpu.VMEM((2 * HH,), _F32),
          pltpu.VMEM((16,), _F32),
          pltpu.VMEM((EPW,), _F32),
          pltpu.VMEM((EPW,), _I32),
      ],
      compiler_params=pltpu.CompilerParams(needs_layout_passes=False),
  )
  return k(a1, a2, b1, b2, srcg, dstg, ib, w2, eb2)


_RB = 1000
_NBLK = NN // _RB


def _tc_b_body(degp, x, wt, b, y_out, dis_out):
  deg = degp[0, :, 0:1] + degp[1, :, 0:1] + 1.0
  dis = lax.rsqrt(deg)
  xw = jnp.dot(x[...], wt[...], preferred_element_type=_F32) + b[...]
  y_out[...] = dis * xw
  dis_out[...] = dis


def _tc_b_call(degp, x, wt, b):
  return pl.pallas_call(
      _tc_b_body,
      grid=(_NBLK,),
      in_specs=[
          pl.BlockSpec((NC, _RB, DEGW), lambda i: (0, i, 0)),
          pl.BlockSpec((_RB, HH), lambda i: (i, 0)),
          pl.BlockSpec((HH, HH), lambda i: (0, 0)),
          pl.BlockSpec((1, HH), lambda i: (0, 0)),
      ],
      out_specs=[
          pl.BlockSpec((_RB, HH), lambda i: (i, 0)),
          pl.BlockSpec((_RB, 1), lambda i: (i, 0)),
      ],
      out_shape=[
          jax.ShapeDtypeStruct((NN, HH), _F32),
          jax.ShapeDtypeStruct((NN, 1), _F32),
      ],
  )(degp, x, wt, b)


def _tc_d_body(accp, y1, dis, bias1, w2t, b2, y2_out):
  d = dis[...]
  h = d * (accp[0] + accp[1] + y1[...]) + bias1[...]
  y2_out[...] = d * (jnp.dot(h, w2t[...], preferred_element_type=_F32) + b2[...])


def _tc_d_call(accp, y1, dis, bias1, w2t, b2):
  return pl.pallas_call(
      _tc_d_body,
      grid=(_NBLK,),
      in_specs=[
          pl.BlockSpec((NC, _RB, HH), lambda i: (0, i, 0)),
          pl.BlockSpec((_RB, HH), lambda i: (i, 0)),
          pl.BlockSpec((_RB, 1), lambda i: (i, 0)),
          pl.BlockSpec((1, HH), lambda i: (0, 0)),
          pl.BlockSpec((HH, HH), lambda i: (0, 0)),
          pl.BlockSpec((1, HH), lambda i: (0, 0)),
      ],
      out_specs=pl.BlockSpec((_RB, HH), lambda i: (i, 0)),
      out_shape=jax.ShapeDtypeStruct((NN, HH), _F32),
  )(accp, y1, dis, bias1, w2t, b2)


def _tc_f_body(accp, y2, dis, bias2, ncw1t, ncb1, ncw2c, ncb2, wa1, wa2, wb1,
               wb2, ecb1a, ecb1b, nl_out, a1_out, a2_out, b1_out, b2_out):
  d = dis[...]
  h = d * (accp[0] + accp[1] + y2[...]) + bias2[...]
  t = jnp.maximum(
      jnp.dot(h, ncw1t[...], preferred_element_type=_F32) + ncb1[...], 0.0)
  nl_out[...] = jnp.dot(t, ncw2c[...], preferred_element_type=_F32) + ncb2[...]
  a1_out[...] = jnp.dot(h, wa1[...], preferred_element_type=_F32) + ecb1a[...]
  a2_out[...] = jnp.dot(h, wa2[...], preferred_element_type=_F32) + ecb1b[...]
  b1_out[...] = jnp.dot(h, wb1[...], preferred_element_type=_F32)
  b2_out[...] = jnp.dot(h, wb2[...], preferred_element_type=_F32)


def _tc_f_call(accp, y2, dis, bias2, ncw1t, ncb1, ncw2c, ncb2, wa1, wa2, wb1,
               wb2, ecb1a, ecb1b):
  full = lambda shp: pl.BlockSpec(shp, lambda i: tuple(0 for _ in shp))
  rowblk = pl.BlockSpec((_RB, HH), lambda i: (i, 0))
  return pl.pallas_call(
      _tc_f_body,
      grid=(_NBLK,),
      in_specs=[
          pl.BlockSpec((NC, _RB, HH), lambda i: (0, i, 0)),
          rowblk,
          pl.BlockSpec((_RB, 1), lambda i: (i, 0)),
          full((1, HH)),
          full((HH, HH)),
          full((1, HH)),
          full((HH, 1)),
          full((1, 1)),
          full((HH, HH)),
          full((HH, HH)),
          full((HH, HH)),
          full((HH, HH)),
          full((1, HH)),
          full((1, HH)),
      ],
      out_specs=[
          pl.BlockSpec((_RB, 1), lambda i: (i, 0)),
          rowblk, rowblk, rowblk, rowblk,
      ],
      out_shape=[
          jax.ShapeDtypeStruct((NN, 1), _F32),
          jax.ShapeDtypeStruct((NN, HH), _F32),
          jax.ShapeDtypeStruct((NN, HH), _F32),
          jax.ShapeDtypeStruct((NN, HH), _F32),
          jax.ShapeDtypeStruct((NN, HH), _F32),
      ],
  )(accp, y2, dis, bias2, ncw1t, ncb1, ncw2c, ncb2, wa1, wa2, wb1, wb2,
    ecb1a, ecb1b)


_NROWS = NPAD16 // 128
_EROWS = EH // 128


def _sm_body(nl2, ib2, el2, eb2, on_out, oe_out):
  nl = nl2[...]
  el = el2[...]
  ib = ib2[...]
  eb = eb2[...]
  m = jnp.maximum(jnp.max(nl), jnp.max(el))
  evn = jnp.exp(nl - m)
  eve = jnp.exp(el - m)
  svals = []
  for b in range(NB):
    sb = (jnp.sum(jnp.where(ib == b, evn, 0.0)) +
          jnp.sum(jnp.where(eb == b, eve, 0.0)))
    svals.append(sb)
  rsn = jnp.zeros_like(nl)
  rse = jnp.zeros_like(el)
  for b in range(NB):
    r = 1.0 / svals[b]
    rsn = rsn + jnp.where(ib == b, r, 0.0)
    rse = rse + jnp.where(eb == b, r, 0.0)
  on_out[...] = evn * rsn
  oe_out[...] = eve * rse


def _sm_call(nl2, ib2, el2, eb2):
  return pl.pallas_call(
      _sm_body,
      out_shape=[
          jax.ShapeDtypeStruct((_NROWS, 128), _F32),
          jax.ShapeDtypeStruct((_EROWS, 128), _F32),
      ],
  )(nl2, ib2, el2, eb2)


def kernel(x, edge_index, edge_attr, info_batch,
           W_g1, b_g1, bias_g1, W_g2, b_g2, bias_g2,
           nc_W1, nc_b1, nc_W2, nc_b2, ec_W1, ec_b1, ec_W2, ec_b2):
  npad = EEP - EE
  rowp = (jnp.arange(npad, dtype=_I32) % 16)
  colp = NN + (jnp.arange(npad, dtype=_I32) % 128)
  rowg = jnp.concatenate([edge_index[0], rowp]).reshape(EGRPS, 128)
  colg = jnp.concatenate([edge_index[1], colp]).reshape(EGRPS, 128)

  degp = _deg_call(colg)

  y1, dis = _tc_b_call(degp, x, W_g1.T, b_g1.reshape(1, HH))
  acc1 = _prop_call(y1, rowg, colg)
  acc1 = _prop_call(y1, rowg, colg)
  y2 = _tc_d_call(acc1, y1, dis, bias_g1.reshape(1, HH), W_g2.T,
                  b_g2.reshape(1, HH))
  acc2 = _prop_call(y2, rowg, colg)

  w1t = ec_W1.T
  nl, a1, a2, b1, b2 = _tc_f_call(
      acc2, y2, dis, bias_g2.reshape(1, HH),
      nc_W1.T, nc_b1.reshape(1, HH), nc_W2.T, nc_b2.reshape(1, 1),
      w1t[:HH, :HH], w1t[:HH, HH:], w1t[HH:, :HH], w1t[HH:, HH:],
      ec_b1[:HH].reshape(1, HH), ec_b1[HH:].reshape(1, HH))

  src = edge_index[0, ::2]
  dst = edge_index[1, ::2]
  padi = (jnp.arange(EHP - EH, dtype=_I32) % 16)
  srcg = jnp.concatenate([src, padi]).reshape(EDGE_ROWS, 128)
  dstg = jnp.concatenate([dst, padi]).reshape(EDGE_ROWS, 128)
  w2 = ec_W2.reshape(2 * HH)
  eb2c = jnp.broadcast_to(ec_b2.reshape(1), (16,))

  elp, ebp = _edge_call(a1, a2, b1, b2, srcg, dstg, info_batch, w2, eb2c)

  nlpad = jnp.concatenate(
      [nl.reshape(NN),
       jnp.full((NPAD16 - NN,), -1e30, _F32)]).reshape(_NROWS, 128)
  ibpad = jnp.concatenate(
      [info_batch, jnp.full((NPAD16 - NN,), NB - 1, _I32)]).reshape(_NROWS, 128)
  el2 = elp[:EH].reshape(_EROWS, 128)
  eb2d = ebp[:EH].reshape(_EROWS, 128)

  on, oe = _sm_call(nlpad, ibpad, el2, eb2d)

  return (on.reshape(NPAD16)[:NN].reshape(NN, 1),
          oe.reshape(EH, 1))
